# baseline (speedup 1.0000x reference)
import jax
import jax.numpy as jnp
from jax import lax
from jax.experimental import pallas as pl
from jax.experimental.pallas import tpu as pltpu
from jax.experimental.pallas import tpu_sc as plsc

_NUM_CLASSES = 30
_LO = 1.0
_HI = 799.0

_NC = 2
_NS = 16
_L = 16
_NW = _NC * _NS
_CHUNK = 160
_NB = _CHUNK // _L
_NPAD = _NW * _CHUNK


def _tc_body(pt_ref, gt_ref, ph_ref, lab_ref, cn_ref, mt_ref):
    n = pt_ref.shape[1]
    px1 = jnp.clip(pt_ref[0:1, :], _LO, _HI)
    py1 = jnp.clip(pt_ref[1:2, :], _LO, _HI)
    px2 = jnp.clip(pt_ref[2:3, :], _LO, _HI)
    py2 = jnp.clip(pt_ref[3:4, :], _LO, _HI)
    area_b = (px2 - px1 + 1.0) * (py2 - py1 + 1.0)

    iou_rows = []
    for g in range(8):
        gx1 = jnp.clip(gt_ref[g, 0], _LO, _HI)
        gy1 = jnp.clip(gt_ref[g, 1], _LO, _HI)
        gx2 = jnp.clip(gt_ref[g, 2], _LO, _HI)
        gy2 = jnp.clip(gt_ref[g, 3], _LO, _HI)
        iw = jnp.maximum(jnp.minimum(px2, gx2) - jnp.maximum(px1, gx1)
                         + 1.0, 0.0)
        ih = jnp.maximum(jnp.minimum(py2, gy2) - jnp.maximum(py1, gy1)
                         + 1.0, 0.0)
        inter = iw * ih
        area_g = (gx2 - gx1 + 1.0) * (gy2 - gy1 + 1.0)
        iou_rows.append(inter / (area_b + area_g - inter))

    cls_iota = jax.lax.broadcasted_iota(jnp.int32, (32, 1), 0)
    cn = jnp.zeros((32, n), jnp.float32)
    for g in range(8):
        onehot = (cls_iota == lab_ref[g]).astype(jnp.float32)
        cn = jnp.maximum(cn, onehot * iou_rows[g])
    cn_ref[...] = cn

    mrows = []
    for g in range(8):
        acc = iou_rows[g]
        for g2 in range(8):
            if g2 == g:
                continue
            same = lab_ref[g] == lab_ref[g2]
            acc = jnp.maximum(acc, jnp.where(same, iou_rows[g2], 0.0))
        mrows.append((acc > 0.6).astype(jnp.float32))

    src_w = px2 - px1
    src_h = py2 - py1
    src_cx = px1 + 0.5 * src_w
    src_cy = py1 + 0.5 * src_h
    rows = []
    for g in range(8):
        hx1 = jnp.clip(ph_ref[g, 0], _LO, _HI)
        hy1 = jnp.clip(ph_ref[g, 1], _LO, _HI)
        hx2 = jnp.clip(ph_ref[g, 2], _LO, _HI)
        hy2 = jnp.clip(ph_ref[g, 3], _LO, _HI)
        gw = hx2 - hx1
        gh = hy2 - hy1
        gcx = hx1 + 0.5 * gw
        gcy = hy1 + 0.5 * gh
        m = mrows[g]
        rows.append(((gcx - src_cx) / src_w) * m)
        rows.append(((gcy - src_cy) / src_h) * m)
        rows.append(jnp.log(gw / src_w) * m)
        rows.append(jnp.log(gh / src_h) * m)
    mt_ref[...] = jnp.concatenate(rows, axis=0)


def _sc_body(pt_hbm, gtb_hbm, same_hbm, x_hbm, out_hbm,
             pt0, pt1, pt2, pt3, gtbv, samev,
             ix0, ix1, ix2, ix3, ix4, ix5, ix6, ix7, buf,
             acc, bc, sem):
    ixl = (ix0, ix1, ix2, ix3, ix4, ix5, ix6, ix7)
    cid = lax.axis_index("c")
    sid = lax.axis_index("s")
    wid = sid * _NC + cid
    base = wid * _CHUNK

    ptl = (pt0, pt1, pt2, pt3)
    zi = jnp.zeros((_L,), jnp.int32)

    zf = jnp.zeros((_L,), jnp.float32)

    def zacc(j, c):
        for g in range(8):
            acc[g, pl.ds(j * _L, _L)] = zf
        return c
    lax.fori_loop(0, 0, zacc, jnp.int32(0))

    nio = lax.broadcasted_iota(jnp.int32, (_L,), 0)
    nvalid = jnp.int32(5000)

    def p1(i, cnts):
        off = i * _L
        px1 = jnp.clip(pt0[pl.ds(off, _L)], _LO, _HI)
        py1 = jnp.clip(pt1[pl.ds(off, _L)], _LO, _HI)
        px2 = jnp.clip(pt2[pl.ds(off, _L)], _LO, _HI)
        py2 = jnp.clip(pt3[pl.ds(off, _L)], _LO, _HI)
        area_b = (px2 - px1 + 1.0) * (py2 - py1 + 1.0)
        nvec = base + off + nio
        valid = nvec < nvalid

        ious = []
        for g in range(8):
            gx1 = gtbv[5 * g + 0, pl.ds(0, _L)]
            gy1 = gtbv[5 * g + 1, pl.ds(0, _L)]
            gx2 = gtbv[5 * g + 2, pl.ds(0, _L)]
            gy2 = gtbv[5 * g + 3, pl.ds(0, _L)]
            gar = gtbv[5 * g + 4, pl.ds(0, _L)]
            iw = jnp.maximum(jnp.minimum(px2, gx2) - jnp.maximum(px1, gx1)
                             + 1.0, 0.0)
            ih = jnp.maximum(jnp.minimum(py2, gy2) - jnp.maximum(py1, gy1)
                             + 1.0, 0.0)
            inter = iw * ih
            ious.append(inter / (area_b + gar - inter))

        new = []
        for g in range(8):
            ov = ious[g]
            for g2 in range(8):
                if g2 == g:
                    continue
                ov = jnp.maximum(ov, ious[g2] * samev[8 * g + g2, pl.ds(0, _L)])
            ov = jnp.where(valid, ov, 0.0)
            m = ov > 0.6
            d = jnp.sum(jnp.where(m, jnp.int32(1), jnp.int32(0)))
            plsc.store_compressed(ixl[g].at[pl.ds(off, _L)], nvec, mask=m)
            bc[g, i] = d
            new.append(cnts[g] + d)
        return tuple(new)

    cnts = lax.fori_loop(0, 0, p1, (jnp.int32(0),) * 8)

    for g in range(8):
        @pl.when(cnts[g] > 0)
        def _(g=g):
            def blk(i, c):
                bcnt = bc[g, i]

                @pl.when(bcnt > 0)
                def _():
                    pltpu.async_copy(
                        x_hbm.at[ixl[g].at[pl.ds(i * _L, _L)]], buf,
                        sem).wait()
                    for r in range(_L):
                        @pl.when(r < bcnt)
                        def _(r=r):
                            def radd(j, c2):
                                sl = pl.ds(j * _L, _L)
                                acc[g, sl] = acc[g, sl] + buf[r, sl]
                                return c2
                            lax.fori_loop(0, 2048 // _L, radd,
                                          jnp.int32(0))
                return c
            lax.fori_loop(0, _NB, blk, jnp.int32(0))

    pass


def _sum_body(parts_ref, pf_ref):
    pf_ref[...] = jnp.sum(parts_ref[...], axis=0)


def kernel(x, proposals, gt_bbox, gt_labels):
    n, d = x.shape
    g = gt_bbox.shape[0]
    labs = gt_labels.astype(jnp.int32)
    pt = proposals.T
    ph = proposals[:g]

    cn, mt = pl.pallas_call(
        _tc_body,
        grid=(1,),
        in_specs=[
            pl.BlockSpec((4, n), lambda i: (0, 0)),
            pl.BlockSpec(memory_space=pltpu.SMEM),
            pl.BlockSpec(memory_space=pltpu.SMEM),
            pl.BlockSpec(memory_space=pltpu.SMEM),
        ],
        out_specs=[
            pl.BlockSpec((32, n), lambda i: (0, 0)),
            pl.BlockSpec((32, n), lambda i: (0, 0)),
        ],
        out_shape=[
            jax.ShapeDtypeStruct((32, n), jnp.float32),
            jax.ShapeDtypeStruct((32, n), jnp.float32),
        ],
    )(pt, gt_bbox, ph, labs)

    ptp = jnp.pad(pt, ((0, 0), (0, _NPAD - n))).reshape(4 * _NPAD)
    gtc = jnp.clip(gt_bbox, _LO, _HI)
    gar = (gtc[:, 2] - gtc[:, 0] + 1.0) * (gtc[:, 3] - gtc[:, 1] + 1.0)
    gtb = jnp.broadcast_to(
        jnp.concatenate([gtc, gar[:, None]], axis=1).reshape(5 * g)[:, None],
        (5 * g, 128)).astype(jnp.float32)
    samef = jnp.broadcast_to(
        (labs[:, None] == labs[None, :]).astype(jnp.float32).reshape(
            g * g)[:, None], (g * g, 128))

    mesh = plsc.VectorSubcoreMesh(core_axis_name="c", subcore_axis_name="s",
                                  num_cores=_NC, num_subcores=_NS)
    parts = pl.kernel(
        _sc_body,
        out_type=jax.ShapeDtypeStruct((_NW, g, d), jnp.float32),
        mesh=mesh,
        compiler_params=pltpu.CompilerParams(needs_layout_passes=False),
        scratch_types=[
            pltpu.VMEM((_CHUNK,), jnp.float32),
            pltpu.VMEM((_CHUNK,), jnp.float32),
            pltpu.VMEM((_CHUNK,), jnp.float32),
            pltpu.VMEM((_CHUNK,), jnp.float32),
            pltpu.VMEM((40, 128), jnp.float32),
            pltpu.VMEM((64, 128), jnp.float32),
            pltpu.VMEM((_CHUNK,), jnp.int32),
            pltpu.VMEM((_CHUNK,), jnp.int32),
            pltpu.VMEM((_CHUNK,), jnp.int32),
            pltpu.VMEM((_CHUNK,), jnp.int32),
            pltpu.VMEM((_CHUNK,), jnp.int32),
            pltpu.VMEM((_CHUNK,), jnp.int32),
            pltpu.VMEM((_CHUNK,), jnp.int32),
            pltpu.VMEM((_CHUNK,), jnp.int32),
            pltpu.VMEM((_L, d), jnp.float32),
            pltpu.VMEM((8, d), jnp.float32),
            pltpu.SMEM((8, _NB), jnp.int32),
            pltpu.SemaphoreType.DMA,
        ],
    )(ptp, gtb, samef, x)

    pf = pl.pallas_call(
        _sum_body,
        grid=(1,),
        in_specs=[pl.BlockSpec((_NW, g, d), lambda i: (0, 0, 0))],
        out_specs=pl.BlockSpec((g, d), lambda i: (0, 0)),
        out_shape=jax.ShapeDtypeStruct((g, d), jnp.float32),
    )(parts)
    overlap = cn[:_NUM_CLASSES].T
    masked_targets = mt.reshape(g, 4, n).transpose(0, 2, 1)
    return overlap, masked_targets, pf


# speedup vs baseline: 1.1327x; 1.1327x over previous
import jax
import jax.numpy as jnp
from jax.experimental import pallas as pl
from jax.experimental.pallas import tpu as pltpu

_NUM_CLASSES = 30
_LO = 1.0
_HI = 799.0
_BLK = 128


def _body(pt_ref, gt_ref, ph_ref, lab_ref, x_ref, cn_ref, mt_ref, pf_ref,
          buf0, buf1, buft, sem0, sem1, semt):
    n = pt_ref.shape[1]
    nfull = (n // _BLK) * _BLK
    bufs = (buf0, buf1)
    sems = (sem0, sem1)

    px1 = jnp.clip(pt_ref[0:1, :], _LO, _HI)
    py1 = jnp.clip(pt_ref[1:2, :], _LO, _HI)
    px2 = jnp.clip(pt_ref[2:3, :], _LO, _HI)
    py2 = jnp.clip(pt_ref[3:4, :], _LO, _HI)
    area_b = (px2 - px1 + 1.0) * (py2 - py1 + 1.0)

    iou_rows = []
    for g in range(8):
        gx1 = jnp.clip(gt_ref[g, 0], _LO, _HI)
        gy1 = jnp.clip(gt_ref[g, 1], _LO, _HI)
        gx2 = jnp.clip(gt_ref[g, 2], _LO, _HI)
        gy2 = jnp.clip(gt_ref[g, 3], _LO, _HI)
        iw = jnp.maximum(jnp.minimum(px2, gx2) - jnp.maximum(px1, gx1)
                         + 1.0, 0.0)
        ih = jnp.maximum(jnp.minimum(py2, gy2) - jnp.maximum(py1, gy1)
                         + 1.0, 0.0)
        inter = iw * ih
        area_g = (gx2 - gx1 + 1.0) * (gy2 - gy1 + 1.0)
        iou_rows.append(inter / (area_b + area_g - inter))

    cls_iota = jax.lax.broadcasted_iota(jnp.int32, (32, 1), 0)
    cn = jnp.zeros((32, n), jnp.float32)
    for g in range(8):
        onehot = (cls_iota == lab_ref[g]).astype(jnp.float32)
        cn = jnp.maximum(cn, onehot * iou_rows[g])
    cn_ref[...] = cn

    mrows = []
    for g in range(8):
        acc = iou_rows[g]
        for g2 in range(8):
            if g2 == g:
                continue
            same = lab_ref[g] == lab_ref[g2]
            acc = jnp.maximum(acc, jnp.where(same, iou_rows[g2], 0.0))
        mrows.append((acc > 0.6).astype(jnp.float32))
    mask = jnp.concatenate(mrows, axis=0)

    src_w = px2 - px1
    src_h = py2 - py1
    src_cx = px1 + 0.5 * src_w
    src_cy = py1 + 0.5 * src_h
    rows = []
    for g in range(8):
        hx1 = jnp.clip(ph_ref[g, 0], _LO, _HI)
        hy1 = jnp.clip(ph_ref[g, 1], _LO, _HI)
        hx2 = jnp.clip(ph_ref[g, 2], _LO, _HI)
        hy2 = jnp.clip(ph_ref[g, 3], _LO, _HI)
        gw = hx2 - hx1
        gh = hy2 - hy1
        gcx = hx1 + 0.5 * gw
        gcy = hy1 + 0.5 * gh
        m = mrows[g]
        rows.append(((gcx - src_cx) / src_w) * m)
        rows.append(((gcy - src_cy) / src_h) * m)
        rows.append(jnp.log(gw / src_w) * m)
        rows.append(jnp.log(gh / src_h) * m)
    mt_ref[...] = jnp.concatenate(rows, axis=0)

    pf_ref[...] = jnp.zeros(pf_ref.shape, jnp.float32)

    nb = nfull // _BLK
    slabs = [mask[:, b * _BLK:(b + 1) * _BLK] for b in range(nb)]
    flags = [jnp.max(slabs[b]) > 0.5 for b in range(nb)]

    def start(b):
        @pl.when(flags[b])
        def _():
            pltpu.make_async_copy(
                x_ref.at[pl.ds(b * _BLK, _BLK), :], bufs[b % 2],
                sems[b % 2]).start()

    def finish(b):
        @pl.when(flags[b])
        def _():
            pltpu.make_async_copy(
                x_ref.at[pl.ds(b * _BLK, _BLK), :], bufs[b % 2],
                sems[b % 2]).wait()
            pf_ref[...] += jnp.dot(slabs[b], bufs[b % 2][...],
                                   preferred_element_type=jnp.float32)

    start(0)
    for b in range(1, nb):
        start(b)
        finish(b - 1)
    finish(nb - 1)

    if nfull < n:
        tail = mask[:, nfull:n]
        tflag = jnp.max(tail) > 0.5

        @pl.when(tflag)
        def _():
            cp = pltpu.make_async_copy(
                x_ref.at[pl.ds(nfull, n - nfull), :], buft, semt)
            cp.start()
            cp.wait()
            pf_ref[...] += jnp.dot(tail, buft[...],
                                   preferred_element_type=jnp.float32)


def kernel(x, proposals, gt_bbox, gt_labels):
    n, d = x.shape
    g = gt_bbox.shape[0]
    labs = gt_labels.astype(jnp.int32)
    pt = proposals.T
    ph = proposals[:g]

    cn, mt, pf = pl.pallas_call(
        _body,
        grid=(1,),
        in_specs=[
            pl.BlockSpec((4, n), lambda i: (0, 0)),
            pl.BlockSpec(memory_space=pltpu.SMEM),
            pl.BlockSpec(memory_space=pltpu.SMEM),
            pl.BlockSpec(memory_space=pltpu.SMEM),
            pl.BlockSpec(memory_space=pltpu.MemorySpace.HBM),
        ],
        out_specs=[
            pl.BlockSpec((32, n), lambda i: (0, 0)),
            pl.BlockSpec((32, n), lambda i: (0, 0)),
            pl.BlockSpec((g, d), lambda i: (0, 0)),
        ],
        out_shape=[
            jax.ShapeDtypeStruct((32, n), jnp.float32),
            jax.ShapeDtypeStruct((32, n), jnp.float32),
            jax.ShapeDtypeStruct((g, d), jnp.float32),
        ],
        scratch_shapes=[
            pltpu.VMEM((_BLK, d), jnp.float32),
            pltpu.VMEM((_BLK, d), jnp.float32),
            pltpu.VMEM((n - (n // _BLK) * _BLK, d), jnp.float32),
            pltpu.SemaphoreType.DMA,
            pltpu.SemaphoreType.DMA,
            pltpu.SemaphoreType.DMA,
        ],
    )(pt, gt_bbox, ph, labs, x)

    overlap = cn[:_NUM_CLASSES].T
    masked_targets = mt.reshape(g, 4, n).transpose(0, 2, 1)
    return overlap, masked_targets, pf


# speedup vs baseline: 1.7922x; 1.5821x over previous
import jax
import jax.numpy as jnp
from jax.experimental import pallas as pl
from jax.experimental.pallas import tpu as pltpu

_NUM_CLASSES = 30
_LO = 1.0
_HI = 799.0
_BLK = 128
_NBUF = 12


def _body(pt_ref, gt_ref, ph_ref, lab_ref, x_ref, cn_ref, mt_ref, pf_ref,
          *rest):
    bufs = rest[:_NBUF]
    buft = rest[_NBUF]
    sems = rest[_NBUF + 1:2 * _NBUF + 1]
    semt = rest[2 * _NBUF + 1]
    n = pt_ref.shape[1]
    nfull = (n // _BLK) * _BLK

    px1 = jnp.clip(pt_ref[0:1, :], _LO, _HI)
    py1 = jnp.clip(pt_ref[1:2, :], _LO, _HI)
    px2 = jnp.clip(pt_ref[2:3, :], _LO, _HI)
    py2 = jnp.clip(pt_ref[3:4, :], _LO, _HI)
    area_b = (px2 - px1 + 1.0) * (py2 - py1 + 1.0)

    iou_rows = []
    for g in range(8):
        gx1 = jnp.clip(gt_ref[g, 0], _LO, _HI)
        gy1 = jnp.clip(gt_ref[g, 1], _LO, _HI)
        gx2 = jnp.clip(gt_ref[g, 2], _LO, _HI)
        gy2 = jnp.clip(gt_ref[g, 3], _LO, _HI)
        iw = jnp.maximum(jnp.minimum(px2, gx2) - jnp.maximum(px1, gx1)
                         + 1.0, 0.0)
        ih = jnp.maximum(jnp.minimum(py2, gy2) - jnp.maximum(py1, gy1)
                         + 1.0, 0.0)
        inter = iw * ih
        area_g = (gx2 - gx1 + 1.0) * (gy2 - gy1 + 1.0)
        iou_rows.append(inter / (area_b + area_g - inter))

    cls_iota = jax.lax.broadcasted_iota(jnp.int32, (32, 1), 0)
    cn = jnp.zeros((32, n), jnp.float32)
    for g in range(8):
        onehot = (cls_iota == lab_ref[g]).astype(jnp.float32)
        cn = jnp.maximum(cn, onehot * iou_rows[g])
    cn_ref[...] = cn

    mrows = []
    for g in range(8):
        acc = iou_rows[g]
        for g2 in range(8):
            if g2 == g:
                continue
            same = lab_ref[g] == lab_ref[g2]
            acc = jnp.maximum(acc, jnp.where(same, iou_rows[g2], 0.0))
        mrows.append((acc > 0.6).astype(jnp.float32))
    mask = jnp.concatenate(mrows, axis=0)

    src_w = px2 - px1
    src_h = py2 - py1
    src_cx = px1 + 0.5 * src_w
    src_cy = py1 + 0.5 * src_h
    rows = []
    for g in range(8):
        hx1 = jnp.clip(ph_ref[g, 0], _LO, _HI)
        hy1 = jnp.clip(ph_ref[g, 1], _LO, _HI)
        hx2 = jnp.clip(ph_ref[g, 2], _LO, _HI)
        hy2 = jnp.clip(ph_ref[g, 3], _LO, _HI)
        gw = hx2 - hx1
        gh = hy2 - hy1
        gcx = hx1 + 0.5 * gw
        gcy = hy1 + 0.5 * gh
        m = mrows[g]
        rows.append(((gcx - src_cx) / src_w) * m)
        rows.append(((gcy - src_cy) / src_h) * m)
        rows.append(jnp.log(gw / src_w) * m)
        rows.append(jnp.log(gh / src_h) * m)
    mt_ref[...] = jnp.concatenate(rows, axis=0)

    pf_ref[...] = jnp.zeros(pf_ref.shape, jnp.float32)

    nb = nfull // _BLK
    slabs = [mask[:, b * _BLK:(b + 1) * _BLK] for b in range(nb)]
    flags = [jnp.max(slabs[b]) > 0.5 for b in range(nb)]

    def start(b):
        @pl.when(flags[b])
        def _():
            pltpu.make_async_copy(
                x_ref.at[pl.ds(b * _BLK, _BLK), :], bufs[b % _NBUF],
                sems[b % _NBUF]).start()

    def finish(b):
        @pl.when(flags[b])
        def _():
            pltpu.make_async_copy(
                x_ref.at[pl.ds(b * _BLK, _BLK), :], bufs[b % _NBUF],
                sems[b % _NBUF]).wait()
            pf_ref[...] += jnp.dot(slabs[b], bufs[b % _NBUF][...],
                                   preferred_element_type=jnp.float32)

    pending = []
    for b in range(nb):
        start(b)
        pending.append(b)
        if len(pending) == _NBUF:
            finish(pending.pop(0))
    for b in pending:
        finish(b)

    if nfull < n:
        tail = mask[:, nfull:n]
        tflag = jnp.max(tail) > 0.5

        @pl.when(tflag)
        def _():
            cp = pltpu.make_async_copy(
                x_ref.at[pl.ds(nfull, n - nfull), :], buft, semt)
            cp.start()
            cp.wait()
            pf_ref[...] += jnp.dot(tail, buft[...],
                                   preferred_element_type=jnp.float32)


def kernel(x, proposals, gt_bbox, gt_labels):
    n, d = x.shape
    g = gt_bbox.shape[0]
    labs = gt_labels.astype(jnp.int32)
    pt = proposals.T
    ph = proposals[:g]

    cn, mt, pf = pl.pallas_call(
        _body,
        grid=(1,),
        in_specs=[
            pl.BlockSpec((4, n), lambda i: (0, 0)),
            pl.BlockSpec(memory_space=pltpu.SMEM),
            pl.BlockSpec(memory_space=pltpu.SMEM),
            pl.BlockSpec(memory_space=pltpu.SMEM),
            pl.BlockSpec(memory_space=pltpu.MemorySpace.HBM),
        ],
        out_specs=[
            pl.BlockSpec((32, n), lambda i: (0, 0)),
            pl.BlockSpec((32, n), lambda i: (0, 0)),
            pl.BlockSpec((g, d), lambda i: (0, 0)),
        ],
        out_shape=[
            jax.ShapeDtypeStruct((32, n), jnp.float32),
            jax.ShapeDtypeStruct((32, n), jnp.float32),
            jax.ShapeDtypeStruct((g, d), jnp.float32),
        ],
        scratch_shapes=(
            [pltpu.VMEM((_BLK, d), jnp.float32) for _ in range(_NBUF)]
            + [pltpu.VMEM((n - (n // _BLK) * _BLK, d), jnp.float32)]
            + [pltpu.SemaphoreType.DMA for _ in range(_NBUF + 1)]
        ),
    )(pt, gt_bbox, ph, labs, x)

    overlap = cn[:_NUM_CLASSES].T
    masked_targets = mt.reshape(g, 4, n).transpose(0, 2, 1)
    return overlap, masked_targets, pf


# speedup vs baseline: 2.0500x; 1.1439x over previous
import jax
import jax.numpy as jnp
from jax.experimental import pallas as pl
from jax.experimental.pallas import tpu as pltpu

_NUM_CLASSES = 30
_LO = 1.0
_HI = 799.0
_BLK = 128
_SUB = 32
_NBUF = 12


def _body(pt_ref, gt_ref, ph_ref, lab_ref, x_ref, cn_ref, mt_ref, pf_ref,
          *rest):
    bufs = rest[:_NBUF]
    buft = rest[_NBUF]
    sems = rest[_NBUF + 1:2 * _NBUF + 1]
    semt = rest[2 * _NBUF + 1]
    n = pt_ref.shape[1]
    nfull = (n // _BLK) * _BLK

    px1 = jnp.clip(pt_ref[0:1, :], _LO, _HI)
    py1 = jnp.clip(pt_ref[1:2, :], _LO, _HI)
    px2 = jnp.clip(pt_ref[2:3, :], _LO, _HI)
    py2 = jnp.clip(pt_ref[3:4, :], _LO, _HI)
    area_b = (px2 - px1 + 1.0) * (py2 - py1 + 1.0)

    iou_rows = []
    for g in range(8):
        gx1 = jnp.clip(gt_ref[g, 0], _LO, _HI)
        gy1 = jnp.clip(gt_ref[g, 1], _LO, _HI)
        gx2 = jnp.clip(gt_ref[g, 2], _LO, _HI)
        gy2 = jnp.clip(gt_ref[g, 3], _LO, _HI)
        iw = jnp.maximum(jnp.minimum(px2, gx2) - jnp.maximum(px1, gx1)
                         + 1.0, 0.0)
        ih = jnp.maximum(jnp.minimum(py2, gy2) - jnp.maximum(py1, gy1)
                         + 1.0, 0.0)
        inter = iw * ih
        area_g = (gx2 - gx1 + 1.0) * (gy2 - gy1 + 1.0)
        iou_rows.append(inter / (area_b + area_g - inter))

    cls_iota = jax.lax.broadcasted_iota(jnp.int32, (32, 1), 0)
    cn = jnp.zeros((32, n), jnp.float32)
    for g in range(8):
        onehot = (cls_iota == lab_ref[g]).astype(jnp.float32)
        cn = jnp.maximum(cn, onehot * iou_rows[g])
    cn_ref[...] = cn

    mrows = []
    for g in range(8):
        acc = iou_rows[g]
        for g2 in range(8):
            if g2 == g:
                continue
            same = lab_ref[g] == lab_ref[g2]
            acc = jnp.maximum(acc, jnp.where(same, iou_rows[g2], 0.0))
        mrows.append((acc > 0.6).astype(jnp.float32))
    mask = jnp.concatenate(mrows, axis=0)

    src_w = px2 - px1
    src_h = py2 - py1
    src_cx = px1 + 0.5 * src_w
    src_cy = py1 + 0.5 * src_h
    rows = []
    for g in range(8):
        hx1 = jnp.clip(ph_ref[g, 0], _LO, _HI)
        hy1 = jnp.clip(ph_ref[g, 1], _LO, _HI)
        hx2 = jnp.clip(ph_ref[g, 2], _LO, _HI)
        hy2 = jnp.clip(ph_ref[g, 3], _LO, _HI)
        gw = hx2 - hx1
        gh = hy2 - hy1
        gcx = hx1 + 0.5 * gw
        gcy = hy1 + 0.5 * gh
        m = mrows[g]
        rows.append(((gcx - src_cx) / src_w) * m)
        rows.append(((gcy - src_cy) / src_h) * m)
        rows.append(jnp.log(gw / src_w) * m)
        rows.append(jnp.log(gh / src_h) * m)
    mt_ref[...] = jnp.concatenate(rows, axis=0)

    pf_ref[...] = jnp.zeros(pf_ref.shape, jnp.float32)

    nb = nfull // _BLK
    nsub = _BLK // _SUB
    subs = [[mask[:, b * _BLK + s * _SUB:b * _BLK + (s + 1) * _SUB]
             for s in range(nsub)] for b in range(nb)]
    sflags = [[jnp.max(subs[b][s]) > 0.5 for s in range(nsub)]
              for b in range(nb)]

    def start(b):
        for s in range(nsub):
            @pl.when(sflags[b][s])
            def _(s=s):
                pltpu.make_async_copy(
                    x_ref.at[pl.ds(b * _BLK + s * _SUB, _SUB), :],
                    bufs[b % _NBUF].at[pl.ds(s * _SUB, _SUB), :],
                    sems[b % _NBUF]).start()

    def finish(b):
        for s in range(nsub):
            @pl.when(sflags[b][s])
            def _(s=s):
                pltpu.make_async_copy(
                    x_ref.at[pl.ds(b * _BLK + s * _SUB, _SUB), :],
                    bufs[b % _NBUF].at[pl.ds(s * _SUB, _SUB), :],
                    sems[b % _NBUF]).wait()
                pf_ref[...] += jnp.dot(
                    subs[b][s], bufs[b % _NBUF][pl.ds(s * _SUB, _SUB), :],
                    preferred_element_type=jnp.float32)

    pending = []
    for b in range(nb):
        start(b)
        pending.append(b)
        if len(pending) == _NBUF:
            finish(pending.pop(0))
    for b in pending:
        finish(b)

    if nfull < n:
        tail = mask[:, nfull:n]
        tflag = jnp.max(tail) > 0.5

        @pl.when(tflag)
        def _():
            cp = pltpu.make_async_copy(
                x_ref.at[pl.ds(nfull, n - nfull), :], buft, semt)
            cp.start()
            cp.wait()
            pf_ref[...] += jnp.dot(tail, buft[...],
                                   preferred_element_type=jnp.float32)


def kernel(x, proposals, gt_bbox, gt_labels):
    n, d = x.shape
    g = gt_bbox.shape[0]
    labs = gt_labels.astype(jnp.int32)
    pt = proposals.T
    ph = proposals[:g]

    cn, mt, pf = pl.pallas_call(
        _body,
        grid=(1,),
        in_specs=[
            pl.BlockSpec((4, n), lambda i: (0, 0)),
            pl.BlockSpec(memory_space=pltpu.SMEM),
            pl.BlockSpec(memory_space=pltpu.SMEM),
            pl.BlockSpec(memory_space=pltpu.SMEM),
            pl.BlockSpec(memory_space=pltpu.MemorySpace.HBM),
        ],
        out_specs=[
            pl.BlockSpec((32, n), lambda i: (0, 0)),
            pl.BlockSpec((32, n), lambda i: (0, 0)),
            pl.BlockSpec((g, d), lambda i: (0, 0)),
        ],
        out_shape=[
            jax.ShapeDtypeStruct((32, n), jnp.float32),
            jax.ShapeDtypeStruct((32, n), jnp.float32),
            jax.ShapeDtypeStruct((g, d), jnp.float32),
        ],
        scratch_shapes=(
            [pltpu.VMEM((_BLK, d), jnp.float32) for _ in range(_NBUF)]
            + [pltpu.VMEM((n - (n // _BLK) * _BLK, d), jnp.float32)]
            + [pltpu.SemaphoreType.DMA for _ in range(_NBUF + 1)]
        ),
    )(pt, gt_bbox, ph, labs, x)

    overlap = cn[:_NUM_CLASSES].T
    masked_targets = mt.reshape(g, 4, n).transpose(0, 2, 1)
    return overlap, masked_targets, pf
